# baseline (speedup 1.0000x reference)
import functools

import jax
import jax.numpy as jnp
from jax import lax
from jax.experimental import pallas as pl
from jax.experimental.pallas import tpu as pltpu
from jax.experimental.pallas import tpu_sc as plsc

_PC = {}

_NW = 32
_K = 80


def _edge_mlp(ea, eW, eb):
    E_, ED_ = ea.shape
    RB = 4000

    def body(ea_ref, w_ref, b_ref, out_ref):
        z = jnp.dot(ea_ref[...], w_ref[...],
                    preferred_element_type=jnp.float32) + b_ref[0]
        out_ref[...] = jnp.maximum(z, 0.0) + jnp.log1p(jnp.exp(-jnp.abs(z)))

    return pl.pallas_call(
        body,
        grid=(E_ // RB,),
        in_specs=[
            pl.BlockSpec((RB, ED_), lambda i: (i, 0)),
            pl.BlockSpec((ED_, 2), lambda i: (0, 0)),
            pl.BlockSpec((1, 2), lambda i: (0, 0)),
        ],
        out_specs=pl.BlockSpec((RB, 2), lambda i: (i, 0)),
        out_shape=jax.ShapeDtypeStruct((E_, 2), jnp.float32),
        **_PC,
    )(ea, eW, eb)


def _wbcast(w_v, flat_idx):
    return plsc.load_gather(w_v, [jnp.full((16,), flat_idx, jnp.int32)])


def _sc_pass(table, src2d, dst2d, w2d, T, N, D):
    E = src2d.size
    EW = E // _NW
    CW = 2000
    NSC = EW // CW
    NBc = CW // _K
    NT = -(-N // (16 * 128)) * 128
    NP = 16 * NT
    TNP = T * NP
    NZ = 64
    G = _K // 16
    mesh = plsc.VectorSubcoreMesh(core_axis_name="c", subcore_axis_name="s")

    srcr = src2d.reshape(_NW * NSC, CW)
    dstr = dst2d.reshape(_NW * NSC, CW)
    wr = w2d.reshape(_NW * NSC, CW)

    assert NBc % 2 == 1
    zeros_nt = jnp.zeros((NT, D), jnp.float32)

    @functools.partial(
        pl.kernel, mesh=mesh,
        compiler_params=pltpu.CompilerParams(needs_layout_passes=False),
        out_type=jax.ShapeDtypeStruct((2 * (T + 1) * NP, D), jnp.float32),
        scratch_types=[
            pltpu.VMEM_SHARED((NP, D), jnp.float32),
            pltpu.VMEM((CW,), jnp.int32),
            pltpu.VMEM((CW,), jnp.int32),
            pltpu.VMEM((CW,), jnp.float32),
            pltpu.VMEM((_K, D), jnp.float32),
            pltpu.VMEM((_K, D), jnp.float32),
            pltpu.VMEM((_K,), jnp.int32),
            pltpu.VMEM((_K,), jnp.int32),
            pltpu.VMEM((_K,), jnp.int32),
            pltpu.VMEM((_K,), jnp.int32),
            pltpu.SemaphoreType.DMA,
            pltpu.SemaphoreType.DMA,
            pltpu.SemaphoreType.DMA,
            pltpu.SemaphoreType.DMA,
        ],
    )
    def kern(table_hbm, src_hbm, dst_hbm, w_hbm, zeros_hbm, part_hbm,
             acc, src_v, dst_v, w_v, rows_a, rows_b, idx_a, idx_b,
             dstb_a, dstb_b, gsem_a, gsem_b, ssem_a, ssem_b):
        c = lax.axis_index("c")
        s = lax.axis_index("s")
        wid = s * 2 + c

        def load_chunk(sc):
            row = wid * NSC + sc
            pltpu.sync_copy(src_hbm.at[row], src_v)
            pltpu.sync_copy(dst_hbm.at[row], dst_v)
            pltpu.sync_copy(w_hbm.at[row], w_v)

        def load_dstb(b, dref):
            for g in range(G):
                dref[pl.ds(g * 16, 16)] = dst_v[pl.ds(b * _K + g * 16, 16)]

        def start_gather(b, t, idx_ref, rows_ref, sem):
            for g in range(G):
                idx_ref[pl.ds(g * 16, 16)] = (
                    src_v[pl.ds(b * _K + g * 16, 16)] + t * N)
            pltpu.async_copy(table_hbm.at[idx_ref], rows_ref, sem)

        def wait_gather(idx_ref, rows_ref, sem):
            pltpu.make_async_copy(table_hbm.at[idx_ref], rows_ref,
                                  sem).wait()

        def wait_scatter(rows_ref, dref, sem):
            pltpu.make_async_copy(rows_ref, acc.at[dref], sem).wait()

        def scale(b, rows_ref):
            def gbody(g, _):
                base = b * _K + g * 16
                for j in range(16):
                    wj = _wbcast(w_v, base + j)
                    row = g * 16 + j
                    for q in range(D // 16):
                        rows_ref[row, pl.ds(q * 16, 16)] = (
                            rows_ref[row, pl.ds(q * 16, 16)] * wj)
                return 0
            lax.fori_loop(0, G, gbody, 0)

        def t_body(t, _):
            pltpu.sync_copy(zeros_hbm, acc.at[pl.ds(s * NT, NT)])
            plsc.subcore_barrier()

            @pl.when(t < T)
            def _gather_phase():
                def chunk_gather(sc, _1):
                    load_chunk(sc)
                    start_gather(0, t, idx_a, rows_a, gsem_a)
                    wait_gather(idx_a, rows_a, gsem_a)
                    scale(0, rows_a)
                    start_gather(1, t, idx_b, rows_b, gsem_b)
                    load_dstb(0, dstb_a)
                    pltpu.async_copy(rows_a, acc.at[dstb_a], ssem_a,
                                     add=True)

                    def pair_body(p, _2):
                        b1 = 2 * p + 1
                        wait_gather(idx_b, rows_b, gsem_b)
                        scale(b1, rows_b)
                        wait_scatter(rows_a, dstb_a, ssem_a)
                        start_gather(b1 + 1, t, idx_a, rows_a, gsem_a)
                        load_dstb(b1, dstb_b)
                        pltpu.async_copy(rows_b, acc.at[dstb_b], ssem_b,
                                         add=True)
                        wait_gather(idx_a, rows_a, gsem_a)
                        scale(b1 + 1, rows_a)
                        wait_scatter(rows_b, dstb_b, ssem_b)

                        @pl.when(b1 + 2 < NBc)
                        def _pf():
                            start_gather(b1 + 2, t, idx_b, rows_b, gsem_b)
                        load_dstb(b1 + 1, dstb_a)
                        pltpu.async_copy(rows_a, acc.at[dstb_a], ssem_a,
                                         add=True)
                        return 0
                    lax.fori_loop(0, (NBc - 1) // 2, pair_body, 0)
                    wait_scatter(rows_a, dstb_a, ssem_a)
                    return 0
                lax.fori_loop(0, NSC, chunk_gather, 0)

            @pl.when(t == T)
            def _deg_phase():
                def chunk_deg(sc, _1):
                    load_chunk(sc)

                    def deg_b(b, _2):
                        def gbody(g, _3):
                            base = b * _K + g * 16
                            for j in range(16):
                                wj = _wbcast(w_v, base + j)
                                row = g * 16 + j
                                for q in range(D // 16):
                                    rows_a[row, pl.ds(q * 16, 16)] = wj
                            return 0
                        lax.fori_loop(0, G, gbody, 0)
                        load_dstb(b, dstb_a)
                        pltpu.sync_copy(rows_a, acc.at[dstb_a], add=True)
                        return 0
                    lax.fori_loop(0, NBc, deg_b, 0)
                    return 0
                lax.fori_loop(0, NSC, chunk_deg, 0)

            plsc.subcore_barrier()
            pltpu.sync_copy(acc.at[pl.ds(s * NT, NT)],
                            part_hbm.at[pl.ds(c * (TNP + NP) + t * NP
                                              + s * NT, NT)])
            plsc.subcore_barrier()
            return 0
        lax.fori_loop(0, T + 1, t_body, 0)

    return kern(table, srcr, dstr, wr, zeros_nt)


def _spmm(table, src2d, dst2d, w2d, T, N, D):
    return _sc_pass(table, src2d, dst2d, w2d, T, N, D)


def _stage_a(part4, x_tnd, W0, b0, g0, bb0):
    T, N, D = x_tnd.shape
    R = 400

    def body(part_ref, degp_ref, x_ref, w_ref, b_ref, g_ref, bb_ref, out_ref):
        p = part_ref[0, 0] + part_ref[1, 0]
        deg = degp_ref[0, 0, :, 0:1] + degp_ref[1, 0, :, 0:1]
        x = x_ref[0]
        aggr = p - deg * x
        z = jnp.dot(aggr, w_ref[...], preferred_element_type=jnp.float32) + b_ref[0]
        r = jnp.maximum(z, 0.0)
        m = jnp.mean(r, axis=-1, keepdims=True)
        v = jnp.mean((r - m) ** 2, axis=-1, keepdims=True)
        out_ref[0] = (r - m) * lax.rsqrt(v + 1e-5) * g_ref[0] + bb_ref[0] + x

    return pl.pallas_call(
        body,
        grid=(T, N // R),
        in_specs=[
            pl.BlockSpec((2, 1, R, D), lambda t, n: (0, t, n, 0)),
            pl.BlockSpec((2, 1, R, D), lambda t, n: (0, T, n, 0)),
            pl.BlockSpec((1, R, D), lambda t, n: (t, n, 0)),
            pl.BlockSpec((D, D), lambda t, n: (0, 0)),
            pl.BlockSpec((1, D), lambda t, n: (0, 0)),
            pl.BlockSpec((1, D), lambda t, n: (0, 0)),
            pl.BlockSpec((1, D), lambda t, n: (0, 0)),
        ],
        out_specs=pl.BlockSpec((1, R, D), lambda t, n: (t, n, 0)),
        out_shape=jax.ShapeDtypeStruct((T, N, D), jnp.float32),
        **_PC,
    )(part4, part4, x_tnd,
      W0, b0.reshape(1, D), g0.reshape(1, D), bb0.reshape(1, D))


def _stage_b(part4, h_tnd, W1, b1, g1, bb1, fcW, fcb,
             WihT, WhhT, bias, decW, decb):
    T, N, D = h_tnd.shape
    RH = WhhT.shape[0]
    R = 400

    def body(part_ref, degp_ref, h_ref, w1_ref, b1_ref, g1_ref, bb1_ref,
             fcw_ref, fcb_ref, wih_ref, whh_ref, bias_ref, dec_ref,
             decb_ref, out_ref):
        deg = degp_ref[0, 0, :, 0:1] + degp_ref[1, 0, :, 0:1]
        hh = jnp.zeros((R, RH), jnp.float32)
        cc = jnp.zeros((R, RH), jnp.float32)
        cols = []
        for t in range(T):
            h1 = h_ref[t]
            aggr = part_ref[0, t] + part_ref[1, t] - deg * h1
            z = jnp.dot(aggr, w1_ref[...],
                        preferred_element_type=jnp.float32) + b1_ref[0]
            r = jnp.maximum(z, 0.0)
            m = jnp.mean(r, axis=-1, keepdims=True)
            v = jnp.mean((r - m) ** 2, axis=-1, keepdims=True)
            h2 = (r - m) * lax.rsqrt(v + 1e-5) * g1_ref[0] + bb1_ref[0] + h1
            emb = jnp.dot(h2, fcw_ref[...],
                          preferred_element_type=jnp.float32) + fcb_ref[0]
            gts = (jnp.dot(emb, wih_ref[...],
                           preferred_element_type=jnp.float32)
                   + jnp.dot(hh, whh_ref[...],
                             preferred_element_type=jnp.float32)
                   + bias_ref[0])
            i_g = jax.nn.sigmoid(gts[:, :RH])
            f_g = jax.nn.sigmoid(gts[:, RH:2 * RH])
            g_g = jnp.tanh(gts[:, 2 * RH:3 * RH])
            o_g = jax.nn.sigmoid(gts[:, 3 * RH:])
            cc = f_g * cc + i_g * g_g
            hh = o_g * jnp.tanh(cc)
            cols.append(jnp.dot(hh, dec_ref[...],
                                preferred_element_type=jnp.float32)
                        + decb_ref[0])
        out_ref[...] = jnp.concatenate(cols, axis=1)

    return pl.pallas_call(
        body,
        grid=(N // R,),
        in_specs=[
            pl.BlockSpec((2, T, R, D), lambda n: (0, 0, n, 0)),
            pl.BlockSpec((2, 1, R, D), lambda n: (0, T, n, 0)),
            pl.BlockSpec((T, R, D), lambda n: (0, n, 0)),
            pl.BlockSpec((D, D), lambda n: (0, 0)),
            pl.BlockSpec((1, D), lambda n: (0, 0)),
            pl.BlockSpec((1, D), lambda n: (0, 0)),
            pl.BlockSpec((1, D), lambda n: (0, 0)),
            pl.BlockSpec((D, D), lambda n: (0, 0)),
            pl.BlockSpec((1, D), lambda n: (0, 0)),
            pl.BlockSpec((D, 4 * RH), lambda n: (0, 0)),
            pl.BlockSpec((RH, 4 * RH), lambda n: (0, 0)),
            pl.BlockSpec((1, 4 * RH), lambda n: (0, 0)),
            pl.BlockSpec((RH, 1), lambda n: (0, 0)),
            pl.BlockSpec((1, 1), lambda n: (0, 0)),
        ],
        out_specs=pl.BlockSpec((R, T), lambda n: (n, 0)),
        out_shape=jax.ShapeDtypeStruct((N, T), jnp.float32),
        **_PC,
    )(part4, part4, h_tnd,
      W1, b1.reshape(1, D), g1.reshape(1, D), bb1.reshape(1, D),
      fcW, fcb.reshape(1, D), WihT, WhhT, bias.reshape(1, 4 * RH),
      decW, decb.reshape(1, 1))


def kernel(X_seq, edge_index, edge_attr, conv0_W, conv0_b, edge0_W,
           edge0_b, ln0_g, ln0_b, conv1_W, conv1_b, edge1_W, edge1_b,
           ln1_g, ln1_b, fc_W, fc_b, W_ih, W_hh, b_ih, b_hh, dec_W, dec_b):
    B, T, N, D = X_seq.shape
    E = edge_index.shape[1]

    eW = jnp.concatenate([edge0_W, edge1_W], axis=1)
    eb = jnp.concatenate([edge0_b, edge1_b]).reshape(1, 2)
    w01 = _edge_mlp(edge_attr, eW, eb)

    src2d = edge_index[0].reshape(E // _K, _K)
    dst2d = edge_index[1].reshape(E // _K, _K)
    w0_2d = w01[:, 0].reshape(E // _K, _K)
    w1_2d = w01[:, 1].reshape(E // _K, _K)

    x_tnd = X_seq.reshape(T, N, D)
    part0 = _spmm(x_tnd.reshape(T * N, D), src2d, dst2d, w0_2d, T, N, D)
    p4_0 = part0.reshape(2, T + 1, -1, D)
    h_tnd = _stage_a(p4_0, x_tnd, conv0_W, conv0_b, ln0_g, ln0_b)
    part1 = _spmm(h_tnd.reshape(T * N, D), src2d, dst2d, w1_2d, T, N, D)
    p4_1 = part1.reshape(2, T + 1, -1, D)
    out = _stage_b(p4_1, h_tnd, conv1_W, conv1_b, ln1_g, ln1_b,
                   fc_W, fc_b, W_ih.T, W_hh.T, b_ih + b_hh, dec_W, dec_b)
    return out.reshape(N, T, 1)


# speedup vs baseline: 1.2971x; 1.2971x over previous
import functools

import jax
import jax.numpy as jnp
from jax import lax
from jax.experimental import pallas as pl
from jax.experimental.pallas import tpu as pltpu
from jax.experimental.pallas import tpu_sc as plsc

_PC = {}

_NW = 32
_K = 80


def _edge_mlp(ea, eW, eb):
    E_, ED_ = ea.shape
    RB = 4000

    def body(ea_ref, w_ref, b_ref, out_ref):
        z = jnp.dot(ea_ref[...], w_ref[...],
                    preferred_element_type=jnp.float32) + b_ref[0]
        out_ref[...] = jnp.maximum(z, 0.0) + jnp.log1p(jnp.exp(-jnp.abs(z)))

    return pl.pallas_call(
        body,
        grid=(E_ // RB,),
        in_specs=[
            pl.BlockSpec((RB, ED_), lambda i: (i, 0)),
            pl.BlockSpec((ED_, 2), lambda i: (0, 0)),
            pl.BlockSpec((1, 2), lambda i: (0, 0)),
        ],
        out_specs=pl.BlockSpec((RB, 2), lambda i: (i, 0)),
        out_shape=jax.ShapeDtypeStruct((E_, 2), jnp.float32),
        **_PC,
    )(ea, eW, eb)


def _wbcast(w_v, flat_idx):
    return plsc.load_gather(w_v, [jnp.full((16,), flat_idx, jnp.int32)])


def _sc_pass(table, src2d, dst2d, w2d, T, N, D):
    E = src2d.size
    EW = E // _NW
    CW = 2000
    NSC = EW // CW
    NBc = CW // _K
    NT = -(-N // (16 * 128)) * 128
    NP = 16 * NT
    TNP = T * NP
    NZ = 64
    G = _K // 16
    mesh = plsc.VectorSubcoreMesh(core_axis_name="c", subcore_axis_name="s")

    srcr = src2d.reshape(_NW * NSC, CW)
    dstr = dst2d.reshape(_NW * NSC, CW)
    wr = w2d.reshape(_NW * NSC, CW)

    assert NBc % 2 == 1
    zeros_nt = jnp.zeros((NT, D), jnp.float32)

    @functools.partial(
        pl.kernel, mesh=mesh,
        compiler_params=pltpu.CompilerParams(needs_layout_passes=False),
        out_type=jax.ShapeDtypeStruct((2 * (T + 1) * NP, D), jnp.float32),
        scratch_types=[
            pltpu.VMEM_SHARED((NP, D), jnp.float32),
            pltpu.VMEM((CW,), jnp.int32),
            pltpu.VMEM((CW,), jnp.int32),
            pltpu.VMEM((CW,), jnp.float32),
            pltpu.VMEM((_K, D), jnp.float32),
            pltpu.VMEM((_K, D), jnp.float32),
            pltpu.VMEM((_K,), jnp.int32),
            pltpu.VMEM((_K,), jnp.int32),
            pltpu.VMEM((_K,), jnp.int32),
            pltpu.VMEM((_K,), jnp.int32),
            pltpu.SemaphoreType.DMA,
            pltpu.SemaphoreType.DMA,
            pltpu.SemaphoreType.DMA,
            pltpu.SemaphoreType.DMA,
        ],
    )
    def kern(table_hbm, src_hbm, dst_hbm, w_hbm, zeros_hbm, part_hbm,
             acc, src_v, dst_v, w_v, rows_a, rows_b, idx_a, idx_b,
             dstb_a, dstb_b, gsem_a, gsem_b, ssem_a, ssem_b):
        c = lax.axis_index("c")
        s = lax.axis_index("s")
        wid = s * 2 + c

        def load_chunk(sc):
            row = wid * NSC + sc
            pltpu.sync_copy(src_hbm.at[row], src_v)
            pltpu.sync_copy(dst_hbm.at[row], dst_v)
            pltpu.sync_copy(w_hbm.at[row], w_v)

        def load_dstb(b, dref):
            for g in range(G):
                dref[pl.ds(g * 16, 16)] = dst_v[pl.ds(b * _K + g * 16, 16)]

        def start_gather(b, t, idx_ref, rows_ref, sem):
            for g in range(G):
                idx_ref[pl.ds(g * 16, 16)] = (
                    src_v[pl.ds(b * _K + g * 16, 16)] + t * N)
            pltpu.async_copy(table_hbm.at[idx_ref], rows_ref, sem)

        def wait_gather(idx_ref, rows_ref, sem):
            pltpu.make_async_copy(table_hbm.at[idx_ref], rows_ref,
                                  sem).wait()

        def wait_scatter(rows_ref, dref, sem):
            pltpu.make_async_copy(rows_ref, acc.at[dref], sem).wait()

        def scale(b, rows_ref):
            def gbody(g, _):
                base = b * _K + g * 16
                for j in range(16):
                    wj = _wbcast(w_v, base + j)
                    row = g * 16 + j
                    for q in range(D // 16):
                        rows_ref[row, pl.ds(q * 16, 16)] = (
                            rows_ref[row, pl.ds(q * 16, 16)] * wj)
                return 0
            lax.fori_loop(0, G, gbody, 0)

        def t_body(t, _):
            pltpu.sync_copy(zeros_hbm, acc.at[pl.ds(s * NT, NT)])
            plsc.subcore_barrier()

            @pl.when(t < T)
            def _gather_phase():
                def run_batch(b, idx_ref, rows_ref, dref, gsem):
                    wait_gather(idx_ref, rows_ref, gsem)
                    scale(b, rows_ref)
                    load_dstb(b, dref)
                    pltpu.sync_copy(rows_ref, acc.at[dref], add=True)

                def chunk_gather(sc, _1):
                    load_chunk(sc)
                    start_gather(0, t, idx_a, rows_a, gsem_a)

                    def pair_body(p, _2):
                        b0 = 2 * p
                        start_gather(b0 + 1, t, idx_b, rows_b, gsem_b)
                        run_batch(b0, idx_a, rows_a, dstb_a, gsem_a)

                        @pl.when(b0 + 2 < NBc)
                        def _pf():
                            start_gather(b0 + 2, t, idx_a, rows_a, gsem_a)
                        run_batch(b0 + 1, idx_b, rows_b, dstb_b, gsem_b)
                        return 0
                    lax.fori_loop(0, NBc // 2, pair_body, 0)
                    run_batch(NBc - 1, idx_a, rows_a, dstb_a, gsem_a)
                    return 0
                lax.fori_loop(0, NSC, chunk_gather, 0)

            @pl.when(t == T)
            def _deg_phase():
                def chunk_deg(sc, _1):
                    load_chunk(sc)

                    def deg_b(b, _2):
                        def gbody(g, _3):
                            base = b * _K + g * 16
                            for j in range(16):
                                wj = _wbcast(w_v, base + j)
                                row = g * 16 + j
                                for q in range(D // 16):
                                    rows_a[row, pl.ds(q * 16, 16)] = wj
                            return 0
                        lax.fori_loop(0, G, gbody, 0)
                        load_dstb(b, dstb_a)
                        pltpu.sync_copy(rows_a, acc.at[dstb_a], add=True)
                        return 0
                    lax.fori_loop(0, NBc, deg_b, 0)
                    return 0
                lax.fori_loop(0, NSC, chunk_deg, 0)

            plsc.subcore_barrier()
            pltpu.sync_copy(acc.at[pl.ds(s * NT, NT)],
                            part_hbm.at[pl.ds(c * (TNP + NP) + t * NP
                                              + s * NT, NT)])
            plsc.subcore_barrier()
            return 0
        lax.fori_loop(0, T + 1, t_body, 0)

    return kern(table, srcr, dstr, wr, zeros_nt)


def _spmm(table, src2d, dst2d, w2d, T, N, D):
    return _sc_pass(table, src2d, dst2d, w2d, T, N, D)


def _stage_a(part4, x_tnd, W0, b0, g0, bb0):
    T, N, D = x_tnd.shape
    R = 400

    def body(part_ref, degp_ref, x_ref, w_ref, b_ref, g_ref, bb_ref, out_ref):
        p = part_ref[0, 0] + part_ref[1, 0]
        deg = degp_ref[0, 0, :, 0:1] + degp_ref[1, 0, :, 0:1]
        x = x_ref[0]
        aggr = p - deg * x
        z = jnp.dot(aggr, w_ref[...], preferred_element_type=jnp.float32) + b_ref[0]
        r = jnp.maximum(z, 0.0)
        m = jnp.mean(r, axis=-1, keepdims=True)
        v = jnp.mean((r - m) ** 2, axis=-1, keepdims=True)
        out_ref[0] = (r - m) * lax.rsqrt(v + 1e-5) * g_ref[0] + bb_ref[0] + x

    return pl.pallas_call(
        body,
        grid=(T, N // R),
        in_specs=[
            pl.BlockSpec((2, 1, R, D), lambda t, n: (0, t, n, 0)),
            pl.BlockSpec((2, 1, R, D), lambda t, n: (0, T, n, 0)),
            pl.BlockSpec((1, R, D), lambda t, n: (t, n, 0)),
            pl.BlockSpec((D, D), lambda t, n: (0, 0)),
            pl.BlockSpec((1, D), lambda t, n: (0, 0)),
            pl.BlockSpec((1, D), lambda t, n: (0, 0)),
            pl.BlockSpec((1, D), lambda t, n: (0, 0)),
        ],
        out_specs=pl.BlockSpec((1, R, D), lambda t, n: (t, n, 0)),
        out_shape=jax.ShapeDtypeStruct((T, N, D), jnp.float32),
        **_PC,
    )(part4, part4, x_tnd,
      W0, b0.reshape(1, D), g0.reshape(1, D), bb0.reshape(1, D))


def _stage_b(part4, h_tnd, W1, b1, g1, bb1, fcW, fcb,
             WihT, WhhT, bias, decW, decb):
    T, N, D = h_tnd.shape
    RH = WhhT.shape[0]
    R = 400

    def body(part_ref, degp_ref, h_ref, w1_ref, b1_ref, g1_ref, bb1_ref,
             fcw_ref, fcb_ref, wih_ref, whh_ref, bias_ref, dec_ref,
             decb_ref, out_ref):
        deg = degp_ref[0, 0, :, 0:1] + degp_ref[1, 0, :, 0:1]
        hh = jnp.zeros((R, RH), jnp.float32)
        cc = jnp.zeros((R, RH), jnp.float32)
        cols = []
        for t in range(T):
            h1 = h_ref[t]
            aggr = part_ref[0, t] + part_ref[1, t] - deg * h1
            z = jnp.dot(aggr, w1_ref[...],
                        preferred_element_type=jnp.float32) + b1_ref[0]
            r = jnp.maximum(z, 0.0)
            m = jnp.mean(r, axis=-1, keepdims=True)
            v = jnp.mean((r - m) ** 2, axis=-1, keepdims=True)
            h2 = (r - m) * lax.rsqrt(v + 1e-5) * g1_ref[0] + bb1_ref[0] + h1
            emb = jnp.dot(h2, fcw_ref[...],
                          preferred_element_type=jnp.float32) + fcb_ref[0]
            gts = (jnp.dot(emb, wih_ref[...],
                           preferred_element_type=jnp.float32)
                   + jnp.dot(hh, whh_ref[...],
                             preferred_element_type=jnp.float32)
                   + bias_ref[0])
            i_g = jax.nn.sigmoid(gts[:, :RH])
            f_g = jax.nn.sigmoid(gts[:, RH:2 * RH])
            g_g = jnp.tanh(gts[:, 2 * RH:3 * RH])
            o_g = jax.nn.sigmoid(gts[:, 3 * RH:])
            cc = f_g * cc + i_g * g_g
            hh = o_g * jnp.tanh(cc)
            cols.append(jnp.dot(hh, dec_ref[...],
                                preferred_element_type=jnp.float32)
                        + decb_ref[0])
        out_ref[...] = jnp.concatenate(cols, axis=1)

    return pl.pallas_call(
        body,
        grid=(N // R,),
        in_specs=[
            pl.BlockSpec((2, T, R, D), lambda n: (0, 0, n, 0)),
            pl.BlockSpec((2, 1, R, D), lambda n: (0, T, n, 0)),
            pl.BlockSpec((T, R, D), lambda n: (0, n, 0)),
            pl.BlockSpec((D, D), lambda n: (0, 0)),
            pl.BlockSpec((1, D), lambda n: (0, 0)),
            pl.BlockSpec((1, D), lambda n: (0, 0)),
            pl.BlockSpec((1, D), lambda n: (0, 0)),
            pl.BlockSpec((D, D), lambda n: (0, 0)),
            pl.BlockSpec((1, D), lambda n: (0, 0)),
            pl.BlockSpec((D, 4 * RH), lambda n: (0, 0)),
            pl.BlockSpec((RH, 4 * RH), lambda n: (0, 0)),
            pl.BlockSpec((1, 4 * RH), lambda n: (0, 0)),
            pl.BlockSpec((RH, 1), lambda n: (0, 0)),
            pl.BlockSpec((1, 1), lambda n: (0, 0)),
        ],
        out_specs=pl.BlockSpec((R, T), lambda n: (n, 0)),
        out_shape=jax.ShapeDtypeStruct((N, T), jnp.float32),
        **_PC,
    )(part4, part4, h_tnd,
      W1, b1.reshape(1, D), g1.reshape(1, D), bb1.reshape(1, D),
      fcW, fcb.reshape(1, D), WihT, WhhT, bias.reshape(1, 4 * RH),
      decW, decb.reshape(1, 1))


def kernel(X_seq, edge_index, edge_attr, conv0_W, conv0_b, edge0_W,
           edge0_b, ln0_g, ln0_b, conv1_W, conv1_b, edge1_W, edge1_b,
           ln1_g, ln1_b, fc_W, fc_b, W_ih, W_hh, b_ih, b_hh, dec_W, dec_b):
    B, T, N, D = X_seq.shape
    E = edge_index.shape[1]

    eW = jnp.concatenate([edge0_W, edge1_W], axis=1)
    eb = jnp.concatenate([edge0_b, edge1_b]).reshape(1, 2)
    w01 = _edge_mlp(edge_attr, eW, eb)

    src2d = edge_index[0].reshape(E // _K, _K)
    dst2d = edge_index[1].reshape(E // _K, _K)
    w0_2d = w01[:, 0].reshape(E // _K, _K)
    w1_2d = w01[:, 1].reshape(E // _K, _K)

    x_tnd = X_seq.reshape(T, N, D)
    part0 = _spmm(x_tnd.reshape(T * N, D), src2d, dst2d, w0_2d, T, N, D)
    p4_0 = part0.reshape(2, T + 1, -1, D)
    h_tnd = _stage_a(p4_0, x_tnd, conv0_W, conv0_b, ln0_g, ln0_b)
    part1 = _spmm(h_tnd.reshape(T * N, D), src2d, dst2d, w1_2d, T, N, D)
    p4_1 = part1.reshape(2, T + 1, -1, D)
    out = _stage_b(p4_1, h_tnd, conv1_W, conv1_b, ln1_g, ln1_b,
                   fc_W, fc_b, W_ih.T, W_hh.T, b_ih + b_hh, dec_W, dec_b)
    return out.reshape(N, T, 1)


# speedup vs baseline: 1.3329x; 1.0276x over previous
import functools

import jax
import jax.numpy as jnp
from jax import lax
from jax.experimental import pallas as pl
from jax.experimental.pallas import tpu as pltpu
from jax.experimental.pallas import tpu_sc as plsc

_PC = {}

_NW = 32
_K = 80


def _edge_mlp(ea, eW, eb):
    E_, ED_ = ea.shape
    RB = 4000

    def body(ea_ref, w_ref, b_ref, out_ref):
        z = jnp.dot(ea_ref[...], w_ref[...],
                    preferred_element_type=jnp.float32) + b_ref[0]
        out_ref[...] = jnp.maximum(z, 0.0) + jnp.log1p(jnp.exp(-jnp.abs(z)))

    return pl.pallas_call(
        body,
        grid=(E_ // RB,),
        in_specs=[
            pl.BlockSpec((RB, ED_), lambda i: (i, 0)),
            pl.BlockSpec((ED_, 2), lambda i: (0, 0)),
            pl.BlockSpec((1, 2), lambda i: (0, 0)),
        ],
        out_specs=pl.BlockSpec((RB, 2), lambda i: (i, 0)),
        out_shape=jax.ShapeDtypeStruct((E_, 2), jnp.float32),
        **_PC,
    )(ea, eW, eb)


def _wbcast(w_v, flat_idx):
    return plsc.load_gather(w_v, [jnp.full((16,), flat_idx, jnp.int32)])


def _sc_pass(table, src2d, dst2d, w2d, T, N, D):
    E = src2d.size
    EW = E // _NW
    CW = 2000
    NSC = EW // CW
    NBc = CW // _K
    NT = -(-N // (16 * 128)) * 128
    NP = 16 * NT
    TNP = T * NP
    NZ = 64
    G = _K // 16
    mesh = plsc.VectorSubcoreMesh(core_axis_name="c", subcore_axis_name="s")

    srcr = src2d.reshape(_NW * NSC, CW)
    dstr = dst2d.reshape(_NW * NSC, CW)
    wr = w2d.reshape(_NW * NSC, CW)

    assert NBc % 2 == 1
    zeros_nt = jnp.zeros((NT, D), jnp.float32)

    @functools.partial(
        pl.kernel, mesh=mesh,
        compiler_params=pltpu.CompilerParams(needs_layout_passes=False),
        out_type=jax.ShapeDtypeStruct((2 * (T + 1) * NP, D), jnp.float32),
        scratch_types=[
            pltpu.VMEM_SHARED((NP, D), jnp.float32),
            pltpu.VMEM((CW,), jnp.int32),
            pltpu.VMEM((CW,), jnp.int32),
            pltpu.VMEM((CW,), jnp.float32),
            pltpu.VMEM((_K, D), jnp.float32),
            pltpu.VMEM((_K, D), jnp.float32),
            pltpu.VMEM((_K,), jnp.int32),
            pltpu.VMEM((_K,), jnp.int32),
            pltpu.VMEM((_K,), jnp.int32),
            pltpu.VMEM((_K,), jnp.int32),
            pltpu.SemaphoreType.DMA,
            pltpu.SemaphoreType.DMA,
            pltpu.SemaphoreType.DMA,
            pltpu.SemaphoreType.DMA,
        ],
    )
    def kern(table_hbm, src_hbm, dst_hbm, w_hbm, zeros_hbm, part_hbm,
             acc, src_v, dst_v, w_v, rows_a, rows_b, idx_a, idx_b,
             dstb_a, dstb_b, gsem_a, gsem_b, ssem_a, ssem_b):
        c = lax.axis_index("c")
        s = lax.axis_index("s")
        wid = s * 2 + c

        def load_chunk(sc):
            row = wid * NSC + sc
            d1 = pltpu.async_copy(src_hbm.at[row], src_v, gsem_a)
            d2 = pltpu.async_copy(dst_hbm.at[row], dst_v, gsem_b)
            d3 = pltpu.async_copy(w_hbm.at[row], w_v, ssem_a)
            d1.wait()
            d2.wait()
            d3.wait()

        def load_dstb(b, dref):
            for g in range(G):
                dref[pl.ds(g * 16, 16)] = dst_v[pl.ds(b * _K + g * 16, 16)]

        def start_gather(b, t, idx_ref, rows_ref, sem):
            for g in range(G):
                idx_ref[pl.ds(g * 16, 16)] = (
                    src_v[pl.ds(b * _K + g * 16, 16)] + t * N)
            pltpu.async_copy(table_hbm.at[idx_ref], rows_ref, sem)

        def wait_gather(idx_ref, rows_ref, sem):
            pltpu.make_async_copy(table_hbm.at[idx_ref], rows_ref,
                                  sem).wait()

        def wait_scatter(rows_ref, dref, sem):
            pltpu.make_async_copy(rows_ref, acc.at[dref], sem).wait()

        def scale(b, rows_ref):
            def gbody(g, _):
                base = b * _K + g * 16
                for j in range(16):
                    wj = _wbcast(w_v, base + j)
                    row = g * 16 + j
                    for q in range(D // 16):
                        rows_ref[row, pl.ds(q * 16, 16)] = (
                            rows_ref[row, pl.ds(q * 16, 16)] * wj)
                return 0
            lax.fori_loop(0, G, gbody, 0)

        def t_body(t, _):
            pltpu.sync_copy(zeros_hbm, acc.at[pl.ds(s * NT, NT)])
            plsc.subcore_barrier()

            @pl.when(t < T)
            def _gather_phase():
                def run_batch(b, idx_ref, rows_ref, dref, gsem):
                    wait_gather(idx_ref, rows_ref, gsem)
                    scale(b, rows_ref)
                    load_dstb(b, dref)
                    pltpu.sync_copy(rows_ref, acc.at[dref], add=True)

                def chunk_gather(sc, _1):
                    load_chunk(sc)
                    start_gather(0, t, idx_a, rows_a, gsem_a)

                    def pair_body(p, _2):
                        b0 = 2 * p
                        start_gather(b0 + 1, t, idx_b, rows_b, gsem_b)
                        run_batch(b0, idx_a, rows_a, dstb_a, gsem_a)

                        @pl.when(b0 + 2 < NBc)
                        def _pf():
                            start_gather(b0 + 2, t, idx_a, rows_a, gsem_a)
                        run_batch(b0 + 1, idx_b, rows_b, dstb_b, gsem_b)
                        return 0
                    lax.fori_loop(0, NBc // 2, pair_body, 0)
                    run_batch(NBc - 1, idx_a, rows_a, dstb_a, gsem_a)
                    return 0
                lax.fori_loop(0, NSC, chunk_gather, 0)

            @pl.when(t == T)
            def _deg_phase():
                def chunk_deg(sc, _1):
                    load_chunk(sc)

                    def deg_b(b, _2):
                        def gbody(g, _3):
                            base = b * _K + g * 16
                            for j in range(16):
                                wj = _wbcast(w_v, base + j)
                                row = g * 16 + j
                                for q in range(D // 16):
                                    rows_a[row, pl.ds(q * 16, 16)] = wj
                            return 0
                        lax.fori_loop(0, G, gbody, 0)
                        load_dstb(b, dstb_a)
                        pltpu.sync_copy(rows_a, acc.at[dstb_a], add=True)
                        return 0
                    lax.fori_loop(0, NBc, deg_b, 0)
                    return 0
                lax.fori_loop(0, NSC, chunk_deg, 0)

            plsc.subcore_barrier()
            pltpu.sync_copy(acc.at[pl.ds(s * NT, NT)],
                            part_hbm.at[pl.ds(c * (TNP + NP) + t * NP
                                              + s * NT, NT)])
            plsc.subcore_barrier()
            return 0
        lax.fori_loop(0, T + 1, t_body, 0)

    return kern(table, srcr, dstr, wr, zeros_nt)


def _spmm(table, src2d, dst2d, w2d, T, N, D):
    return _sc_pass(table, src2d, dst2d, w2d, T, N, D)


def _stage_a(part4, x_tnd, W0, b0, g0, bb0):
    T, N, D = x_tnd.shape
    R = 400

    def body(part_ref, degp_ref, x_ref, w_ref, b_ref, g_ref, bb_ref, out_ref):
        p = part_ref[0, 0] + part_ref[1, 0]
        deg = degp_ref[0, 0, :, 0:1] + degp_ref[1, 0, :, 0:1]
        x = x_ref[0]
        aggr = p - deg * x
        z = jnp.dot(aggr, w_ref[...], preferred_element_type=jnp.float32) + b_ref[0]
        r = jnp.maximum(z, 0.0)
        m = jnp.mean(r, axis=-1, keepdims=True)
        v = jnp.mean((r - m) ** 2, axis=-1, keepdims=True)
        out_ref[0] = (r - m) * lax.rsqrt(v + 1e-5) * g_ref[0] + bb_ref[0] + x

    return pl.pallas_call(
        body,
        grid=(T, N // R),
        in_specs=[
            pl.BlockSpec((2, 1, R, D), lambda t, n: (0, t, n, 0)),
            pl.BlockSpec((2, 1, R, D), lambda t, n: (0, T, n, 0)),
            pl.BlockSpec((1, R, D), lambda t, n: (t, n, 0)),
            pl.BlockSpec((D, D), lambda t, n: (0, 0)),
            pl.BlockSpec((1, D), lambda t, n: (0, 0)),
            pl.BlockSpec((1, D), lambda t, n: (0, 0)),
            pl.BlockSpec((1, D), lambda t, n: (0, 0)),
        ],
        out_specs=pl.BlockSpec((1, R, D), lambda t, n: (t, n, 0)),
        out_shape=jax.ShapeDtypeStruct((T, N, D), jnp.float32),
        **_PC,
    )(part4, part4, x_tnd,
      W0, b0.reshape(1, D), g0.reshape(1, D), bb0.reshape(1, D))


def _stage_b(part4, h_tnd, W1, b1, g1, bb1, fcW, fcb,
             WihT, WhhT, bias, decW, decb):
    T, N, D = h_tnd.shape
    RH = WhhT.shape[0]
    R = 400

    def body(part_ref, degp_ref, h_ref, w1_ref, b1_ref, g1_ref, bb1_ref,
             fcw_ref, fcb_ref, wih_ref, whh_ref, bias_ref, dec_ref,
             decb_ref, out_ref):
        deg = degp_ref[0, 0, :, 0:1] + degp_ref[1, 0, :, 0:1]
        hh = jnp.zeros((R, RH), jnp.float32)
        cc = jnp.zeros((R, RH), jnp.float32)
        cols = []
        for t in range(T):
            h1 = h_ref[t]
            aggr = part_ref[0, t] + part_ref[1, t] - deg * h1
            z = jnp.dot(aggr, w1_ref[...],
                        preferred_element_type=jnp.float32) + b1_ref[0]
            r = jnp.maximum(z, 0.0)
            m = jnp.mean(r, axis=-1, keepdims=True)
            v = jnp.mean((r - m) ** 2, axis=-1, keepdims=True)
            h2 = (r - m) * lax.rsqrt(v + 1e-5) * g1_ref[0] + bb1_ref[0] + h1
            emb = jnp.dot(h2, fcw_ref[...],
                          preferred_element_type=jnp.float32) + fcb_ref[0]
            gts = (jnp.dot(emb, wih_ref[...],
                           preferred_element_type=jnp.float32)
                   + jnp.dot(hh, whh_ref[...],
                             preferred_element_type=jnp.float32)
                   + bias_ref[0])
            i_g = jax.nn.sigmoid(gts[:, :RH])
            f_g = jax.nn.sigmoid(gts[:, RH:2 * RH])
            g_g = jnp.tanh(gts[:, 2 * RH:3 * RH])
            o_g = jax.nn.sigmoid(gts[:, 3 * RH:])
            cc = f_g * cc + i_g * g_g
            hh = o_g * jnp.tanh(cc)
            cols.append(jnp.dot(hh, dec_ref[...],
                                preferred_element_type=jnp.float32)
                        + decb_ref[0])
        out_ref[...] = jnp.concatenate(cols, axis=1)

    return pl.pallas_call(
        body,
        grid=(N // R,),
        in_specs=[
            pl.BlockSpec((2, T, R, D), lambda n: (0, 0, n, 0)),
            pl.BlockSpec((2, 1, R, D), lambda n: (0, T, n, 0)),
            pl.BlockSpec((T, R, D), lambda n: (0, n, 0)),
            pl.BlockSpec((D, D), lambda n: (0, 0)),
            pl.BlockSpec((1, D), lambda n: (0, 0)),
            pl.BlockSpec((1, D), lambda n: (0, 0)),
            pl.BlockSpec((1, D), lambda n: (0, 0)),
            pl.BlockSpec((D, D), lambda n: (0, 0)),
            pl.BlockSpec((1, D), lambda n: (0, 0)),
            pl.BlockSpec((D, 4 * RH), lambda n: (0, 0)),
            pl.BlockSpec((RH, 4 * RH), lambda n: (0, 0)),
            pl.BlockSpec((1, 4 * RH), lambda n: (0, 0)),
            pl.BlockSpec((RH, 1), lambda n: (0, 0)),
            pl.BlockSpec((1, 1), lambda n: (0, 0)),
        ],
        out_specs=pl.BlockSpec((R, T), lambda n: (n, 0)),
        out_shape=jax.ShapeDtypeStruct((N, T), jnp.float32),
        **_PC,
    )(part4, part4, h_tnd,
      W1, b1.reshape(1, D), g1.reshape(1, D), bb1.reshape(1, D),
      fcW, fcb.reshape(1, D), WihT, WhhT, bias.reshape(1, 4 * RH),
      decW, decb.reshape(1, 1))


def kernel(X_seq, edge_index, edge_attr, conv0_W, conv0_b, edge0_W,
           edge0_b, ln0_g, ln0_b, conv1_W, conv1_b, edge1_W, edge1_b,
           ln1_g, ln1_b, fc_W, fc_b, W_ih, W_hh, b_ih, b_hh, dec_W, dec_b):
    B, T, N, D = X_seq.shape
    E = edge_index.shape[1]

    eW = jnp.concatenate([edge0_W, edge1_W], axis=1)
    eb = jnp.concatenate([edge0_b, edge1_b]).reshape(1, 2)
    w01 = _edge_mlp(edge_attr, eW, eb)

    src2d = edge_index[0].reshape(E // _K, _K)
    dst2d = edge_index[1].reshape(E // _K, _K)
    w0_2d = w01[:, 0].reshape(E // _K, _K)
    w1_2d = w01[:, 1].reshape(E // _K, _K)

    x_tnd = X_seq.reshape(T, N, D)
    part0 = _spmm(x_tnd.reshape(T * N, D), src2d, dst2d, w0_2d, T, N, D)
    p4_0 = part0.reshape(2, T + 1, -1, D)
    h_tnd = _stage_a(p4_0, x_tnd, conv0_W, conv0_b, ln0_g, ln0_b)
    part1 = _spmm(h_tnd.reshape(T * N, D), src2d, dst2d, w1_2d, T, N, D)
    p4_1 = part1.reshape(2, T + 1, -1, D)
    out = _stage_b(p4_1, h_tnd, conv1_W, conv1_b, ln1_g, ln1_b,
                   fc_W, fc_b, W_ih.T, W_hh.T, b_ih + b_hh, dec_W, dec_b)
    return out.reshape(N, T, 1)


# speedup vs baseline: 1.3343x; 1.0010x over previous
import functools

import jax
import jax.numpy as jnp
from jax import lax
from jax.experimental import pallas as pl
from jax.experimental.pallas import tpu as pltpu
from jax.experimental.pallas import tpu_sc as plsc

_NW = 32
_K = 80


def _edge_mlp(ea, eW, eb):
    E_, ED_ = ea.shape
    RB = 4000

    def body(ea_ref, w_ref, b_ref, out_ref):
        z = jnp.dot(ea_ref[...], w_ref[...],
                    preferred_element_type=jnp.float32) + b_ref[0]
        out_ref[...] = jnp.maximum(z, 0.0) + jnp.log1p(jnp.exp(-jnp.abs(z)))

    return pl.pallas_call(
        body,
        grid=(E_ // RB,),
        in_specs=[
            pl.BlockSpec((RB, ED_), lambda i: (i, 0)),
            pl.BlockSpec((ED_, 2), lambda i: (0, 0)),
            pl.BlockSpec((1, 2), lambda i: (0, 0)),
        ],
        out_specs=pl.BlockSpec((RB, 2), lambda i: (i, 0)),
        out_shape=jax.ShapeDtypeStruct((E_, 2), jnp.float32),
    )(ea, eW, eb)


def _wbcast(w_v, flat_idx):
    return plsc.load_gather(w_v, [jnp.full((16,), flat_idx, jnp.int32)])


def _sc_pass(table, src2d, dst2d, w2d, T, N, D):
    E = src2d.size
    EW = E // _NW
    CW = 2000
    NSC = EW // CW
    NBc = CW // _K
    NT = -(-N // (16 * 128)) * 128
    NP = 16 * NT
    TNP = T * NP
    NZ = 64
    G = _K // 16
    mesh = plsc.VectorSubcoreMesh(core_axis_name="c", subcore_axis_name="s")

    srcr = src2d.reshape(_NW * NSC, CW)
    dstr = dst2d.reshape(_NW * NSC, CW)
    wr = w2d.reshape(_NW * NSC, CW)

    assert NBc % 2 == 1
    zeros_nt = jnp.zeros((NT, D), jnp.float32)

    @functools.partial(
        pl.kernel, mesh=mesh,
        compiler_params=pltpu.CompilerParams(needs_layout_passes=False),
        out_type=jax.ShapeDtypeStruct((2 * (T + 1) * NP, D), jnp.float32),
        scratch_types=[
            pltpu.VMEM_SHARED((NP, D), jnp.float32),
            pltpu.VMEM((CW,), jnp.int32),
            pltpu.VMEM((CW,), jnp.int32),
            pltpu.VMEM((CW,), jnp.float32),
            pltpu.VMEM((_K, D), jnp.float32),
            pltpu.VMEM((_K, D), jnp.float32),
            pltpu.VMEM((_K,), jnp.int32),
            pltpu.VMEM((_K,), jnp.int32),
            pltpu.VMEM((_K,), jnp.int32),
            pltpu.VMEM((_K,), jnp.int32),
            pltpu.SemaphoreType.DMA,
            pltpu.SemaphoreType.DMA,
            pltpu.SemaphoreType.DMA,
            pltpu.SemaphoreType.DMA,
        ],
    )
    def kern(table_hbm, src_hbm, dst_hbm, w_hbm, zeros_hbm, part_hbm,
             acc, src_v, dst_v, w_v, rows_a, rows_b, idx_a, idx_b,
             dstb_a, dstb_b, gsem_a, gsem_b, ssem_a, ssem_b):
        c = lax.axis_index("c")
        s = lax.axis_index("s")
        wid = s * 2 + c

        def load_chunk(sc):
            row = wid * NSC + sc
            d1 = pltpu.async_copy(src_hbm.at[row], src_v, gsem_a)
            d2 = pltpu.async_copy(dst_hbm.at[row], dst_v, gsem_b)
            d3 = pltpu.async_copy(w_hbm.at[row], w_v, ssem_a)
            d1.wait()
            d2.wait()
            d3.wait()

        def load_dstb(b, dref):
            for g in range(G):
                dref[pl.ds(g * 16, 16)] = dst_v[pl.ds(b * _K + g * 16, 16)]

        def start_gather(b, t, idx_ref, rows_ref, sem):
            for g in range(G):
                idx_ref[pl.ds(g * 16, 16)] = (
                    src_v[pl.ds(b * _K + g * 16, 16)] + t * N)
            pltpu.async_copy(table_hbm.at[idx_ref], rows_ref, sem)

        def wait_gather(idx_ref, rows_ref, sem):
            pltpu.make_async_copy(table_hbm.at[idx_ref], rows_ref,
                                  sem).wait()

        def wait_scatter(rows_ref, dref, sem):
            pltpu.make_async_copy(rows_ref, acc.at[dref], sem).wait()

        def scale(b, rows_ref):
            def gbody(g, _):
                base = b * _K + g * 16
                for j in range(16):
                    wj = _wbcast(w_v, base + j)
                    row = g * 16 + j
                    for q in range(D // 16):
                        rows_ref[row, pl.ds(q * 16, 16)] = (
                            rows_ref[row, pl.ds(q * 16, 16)] * wj)
                return 0
            lax.fori_loop(0, G, gbody, 0)

        def t_body(t, _):
            pltpu.sync_copy(zeros_hbm, acc.at[pl.ds(s * NT, NT)])
            plsc.subcore_barrier()

            @pl.when(t < T)
            def _gather_phase():
                def run_batch(b, idx_ref, rows_ref, dref, gsem):
                    wait_gather(idx_ref, rows_ref, gsem)
                    scale(b, rows_ref)
                    load_dstb(b, dref)
                    pltpu.sync_copy(rows_ref, acc.at[dref], add=True)

                def chunk_gather(sc, _1):
                    load_chunk(sc)
                    start_gather(0, t, idx_a, rows_a, gsem_a)

                    def pair_body(p, _2):
                        b0 = 2 * p
                        start_gather(b0 + 1, t, idx_b, rows_b, gsem_b)
                        run_batch(b0, idx_a, rows_a, dstb_a, gsem_a)

                        @pl.when(b0 + 2 < NBc)
                        def _pf():
                            start_gather(b0 + 2, t, idx_a, rows_a, gsem_a)
                        run_batch(b0 + 1, idx_b, rows_b, dstb_b, gsem_b)
                        return 0
                    lax.fori_loop(0, NBc // 2, pair_body, 0)
                    run_batch(NBc - 1, idx_a, rows_a, dstb_a, gsem_a)
                    return 0
                lax.fori_loop(0, NSC, chunk_gather, 0)

            @pl.when(t == T)
            def _deg_phase():
                def chunk_deg(sc, _1):
                    load_chunk(sc)

                    def deg_b(b, _2):
                        def gbody(g, _3):
                            base = b * _K + g * 16
                            for j in range(16):
                                wj = _wbcast(w_v, base + j)
                                row = g * 16 + j
                                for q in range(D // 16):
                                    rows_a[row, pl.ds(q * 16, 16)] = wj
                            return 0
                        lax.fori_loop(0, G, gbody, 0)
                        load_dstb(b, dstb_a)
                        pltpu.sync_copy(rows_a, acc.at[dstb_a], add=True)
                        return 0
                    lax.fori_loop(0, NBc, deg_b, 0)
                    return 0
                lax.fori_loop(0, NSC, chunk_deg, 0)

            plsc.subcore_barrier()
            pltpu.sync_copy(acc.at[pl.ds(s * NT, NT)],
                            part_hbm.at[pl.ds(c * (TNP + NP) + t * NP
                                              + s * NT, NT)])
            plsc.subcore_barrier()
            return 0
        lax.fori_loop(0, T + 1, t_body, 0)

    return kern(table, srcr, dstr, wr, zeros_nt)


def _spmm(table, src2d, dst2d, w2d, T, N, D):
    return _sc_pass(table, src2d, dst2d, w2d, T, N, D)


def _stage_a(part4, x_tnd, W0, b0, g0, bb0):
    T, N, D = x_tnd.shape
    R = 400

    def body(part_ref, degp_ref, x_ref, w_ref, b_ref, g_ref, bb_ref, out_ref):
        p = part_ref[0, 0] + part_ref[1, 0]
        deg = degp_ref[0, 0, :, 0:1] + degp_ref[1, 0, :, 0:1]
        x = x_ref[0]
        aggr = p - deg * x
        z = jnp.dot(aggr, w_ref[...], preferred_element_type=jnp.float32) + b_ref[0]
        r = jnp.maximum(z, 0.0)
        m = jnp.mean(r, axis=-1, keepdims=True)
        v = jnp.mean((r - m) ** 2, axis=-1, keepdims=True)
        out_ref[0] = (r - m) * lax.rsqrt(v + 1e-5) * g_ref[0] + bb_ref[0] + x

    return pl.pallas_call(
        body,
        grid=(T, N // R),
        in_specs=[
            pl.BlockSpec((2, 1, R, D), lambda t, n: (0, t, n, 0)),
            pl.BlockSpec((2, 1, R, D), lambda t, n: (0, T, n, 0)),
            pl.BlockSpec((1, R, D), lambda t, n: (t, n, 0)),
            pl.BlockSpec((D, D), lambda t, n: (0, 0)),
            pl.BlockSpec((1, D), lambda t, n: (0, 0)),
            pl.BlockSpec((1, D), lambda t, n: (0, 0)),
            pl.BlockSpec((1, D), lambda t, n: (0, 0)),
        ],
        out_specs=pl.BlockSpec((1, R, D), lambda t, n: (t, n, 0)),
        out_shape=jax.ShapeDtypeStruct((T, N, D), jnp.float32),
    )(part4, part4, x_tnd,
      W0, b0.reshape(1, D), g0.reshape(1, D), bb0.reshape(1, D))


def _stage_b(part4, h_tnd, W1, b1, g1, bb1, fcW, fcb,
             WihT, WhhT, bias, decW, decb):
    T, N, D = h_tnd.shape
    RH = WhhT.shape[0]
    R = 400

    def body(part_ref, degp_ref, h_ref, w1_ref, b1_ref, g1_ref, bb1_ref,
             fcw_ref, fcb_ref, wih_ref, whh_ref, bias_ref, dec_ref,
             decb_ref, out_ref):
        deg = degp_ref[0, 0, :, 0:1] + degp_ref[1, 0, :, 0:1]
        hh = jnp.zeros((R, RH), jnp.float32)
        cc = jnp.zeros((R, RH), jnp.float32)
        cols = []
        for t in range(T):
            h1 = h_ref[t]
            aggr = part_ref[0, t] + part_ref[1, t] - deg * h1
            z = jnp.dot(aggr, w1_ref[...],
                        preferred_element_type=jnp.float32) + b1_ref[0]
            r = jnp.maximum(z, 0.0)
            m = jnp.mean(r, axis=-1, keepdims=True)
            v = jnp.mean((r - m) ** 2, axis=-1, keepdims=True)
            h2 = (r - m) * lax.rsqrt(v + 1e-5) * g1_ref[0] + bb1_ref[0] + h1
            emb = jnp.dot(h2, fcw_ref[...],
                          preferred_element_type=jnp.float32) + fcb_ref[0]
            gts = (jnp.dot(emb, wih_ref[...],
                           preferred_element_type=jnp.float32)
                   + jnp.dot(hh, whh_ref[...],
                             preferred_element_type=jnp.float32)
                   + bias_ref[0])
            i_g = jax.nn.sigmoid(gts[:, :RH])
            f_g = jax.nn.sigmoid(gts[:, RH:2 * RH])
            g_g = jnp.tanh(gts[:, 2 * RH:3 * RH])
            o_g = jax.nn.sigmoid(gts[:, 3 * RH:])
            cc = f_g * cc + i_g * g_g
            hh = o_g * jnp.tanh(cc)
            cols.append(jnp.dot(hh, dec_ref[...],
                                preferred_element_type=jnp.float32)
                        + decb_ref[0])
        out_ref[...] = jnp.concatenate(cols, axis=1)

    return pl.pallas_call(
        body,
        grid=(N // R,),
        in_specs=[
            pl.BlockSpec((2, T, R, D), lambda n: (0, 0, n, 0)),
            pl.BlockSpec((2, 1, R, D), lambda n: (0, T, n, 0)),
            pl.BlockSpec((T, R, D), lambda n: (0, n, 0)),
            pl.BlockSpec((D, D), lambda n: (0, 0)),
            pl.BlockSpec((1, D), lambda n: (0, 0)),
            pl.BlockSpec((1, D), lambda n: (0, 0)),
            pl.BlockSpec((1, D), lambda n: (0, 0)),
            pl.BlockSpec((D, D), lambda n: (0, 0)),
            pl.BlockSpec((1, D), lambda n: (0, 0)),
            pl.BlockSpec((D, 4 * RH), lambda n: (0, 0)),
            pl.BlockSpec((RH, 4 * RH), lambda n: (0, 0)),
            pl.BlockSpec((1, 4 * RH), lambda n: (0, 0)),
            pl.BlockSpec((RH, 1), lambda n: (0, 0)),
            pl.BlockSpec((1, 1), lambda n: (0, 0)),
        ],
        out_specs=pl.BlockSpec((R, T), lambda n: (n, 0)),
        out_shape=jax.ShapeDtypeStruct((N, T), jnp.float32),
    )(part4, part4, h_tnd,
      W1, b1.reshape(1, D), g1.reshape(1, D), bb1.reshape(1, D),
      fcW, fcb.reshape(1, D), WihT, WhhT, bias.reshape(1, 4 * RH),
      decW, decb.reshape(1, 1))


def kernel(X_seq, edge_index, edge_attr, conv0_W, conv0_b, edge0_W,
           edge0_b, ln0_g, ln0_b, conv1_W, conv1_b, edge1_W, edge1_b,
           ln1_g, ln1_b, fc_W, fc_b, W_ih, W_hh, b_ih, b_hh, dec_W, dec_b):
    B, T, N, D = X_seq.shape
    E = edge_index.shape[1]

    eW = jnp.concatenate([edge0_W, edge1_W], axis=1)
    eb = jnp.concatenate([edge0_b, edge1_b]).reshape(1, 2)
    w01 = _edge_mlp(edge_attr, eW, eb)

    src2d = edge_index[0].reshape(E // _K, _K)
    dst2d = edge_index[1].reshape(E // _K, _K)
    w0_2d = w01[:, 0].reshape(E // _K, _K)
    w1_2d = w01[:, 1].reshape(E // _K, _K)

    x_tnd = X_seq.reshape(T, N, D)
    part0 = _spmm(x_tnd.reshape(T * N, D), src2d, dst2d, w0_2d, T, N, D)
    p4_0 = part0.reshape(2, T + 1, -1, D)
    h_tnd = _stage_a(p4_0, x_tnd, conv0_W, conv0_b, ln0_g, ln0_b)
    part1 = _spmm(h_tnd.reshape(T * N, D), src2d, dst2d, w1_2d, T, N, D)
    p4_1 = part1.reshape(2, T + 1, -1, D)
    out = _stage_b(p4_1, h_tnd, conv1_W, conv1_b, ln1_g, ln1_b,
                   fc_W, fc_b, W_ih.T, W_hh.T, b_ih + b_hh, dec_W, dec_b)
    return out.reshape(N, T, 1)
